# baseline (speedup 1.0000x reference)
import functools

import jax
import jax.numpy as jnp
from jax import lax
from jax.experimental import pallas as pl
from jax.experimental.pallas import tpu as pltpu
from jax.experimental.pallas import tpu_sc as plsc

F32 = jnp.float32
BF16 = jnp.bfloat16
EPS = 1e-5
NCORES, NSUB = 2, 16
NWORK = NCORES * NSUB

BN = 2000
BE = 2000
BF = 400


def _pack2(pa, pb):
    ua = lax.bitcast_convert_type(pa, jnp.uint32)
    ub = lax.bitcast_convert_type(pb, jnp.uint32)
    ra = (ua + ((ua >> 16) & 1) + 0x7FFF) >> 16
    rb = (ub + ((ub >> 16) & 1) + 0x7FFF) & jnp.uint32(0xFFFF0000)
    return lax.bitcast_convert_type(ra | rb, F32)


def _unpack2(g):
    u = lax.bitcast_convert_type(g, jnp.uint32)
    lo = lax.bitcast_convert_type(u << 16, F32)
    hi = lax.bitcast_convert_type(u & jnp.uint32(0xFFFF0000), F32)
    return lo, hi


def _ln(h, g, be):
    m = jnp.mean(h, axis=-1, keepdims=True)
    v = jnp.mean((h - m) ** 2, axis=-1, keepdims=True)
    return (h - m) * lax.rsqrt(v + EPS) * g + be


def _mm(a, b):
    return jnp.dot(a, b, preferred_element_type=F32)


def _proj_pack(node, ws_lo, ws_hi):
    return _pack2(_mm(node, ws_lo[...]), _mm(node, ws_hi[...]))


def _enc_node_body(x_r, w1, b1, w2, b2, w3, b3, g, be,
                   ws_lo, ws_hi, wd_lo, wd_hi, node_o, ps_o, pd_o):
    h = jnp.maximum(_mm(x_r[...], w1[...]) + b1[...], 0.0)
    h = jnp.maximum(_mm(h, w2[...]) + b2[...], 0.0)
    h = _mm(h, w3[...]) + b3[...]
    node = _ln(h, g[...], be[...])
    node_o[...] = node
    ps_o[...] = _proj_pack(node, ws_lo, ws_hi)
    pd_o[...] = _proj_pack(node, wd_lo, wd_hi)


def _enc_edge_body(x_r, w1, b1, w2, b2, w3, b3, g, be, edge_o):
    h = jnp.maximum(_mm(x_r[...].astype(BF16), w1[...]) + b1[...], 0.0)
    h = jnp.maximum(_mm(h.astype(BF16), w2[...]) + b2[...], 0.0)
    h = _mm(h.astype(BF16), w3[...]) + b3[...]
    edge_o[...] = _ln(h, g[...], be[...]).astype(BF16)


def _edge_mlp_body(with_new, edge_r, g_r, w1a, m1, m2, b1, w2, b2, w3, b3,
                   g, be, *outs):
    u = lax.bitcast_convert_type(g_r[...], jnp.uint32)
    lo = lax.bitcast_convert_type(u << 16, F32).astype(BF16)
    hi = lax.bitcast_convert_type(u & jnp.uint32(0xFFFF0000), F32).astype(BF16)
    h1 = jnp.maximum(_mm(edge_r[...], w1a[...]) + _mm(lo, m1[...])
                     + _mm(hi, m2[...]) + b1[...], 0.0)
    h2 = jnp.maximum(_mm(h1.astype(BF16), w2[...]) + b2[...], 0.0)
    h3 = _mm(h2.astype(BF16), w3[...]) + b3[...]
    upd = _ln(h3, g[...], be[...])
    outs[0][...] = upd
    if with_new:
        outs[1][...] = (edge_r[...].astype(F32) + upd).astype(BF16)


def _node_mlp_body(with_proj, node_r, a0_r, a1_r, v1a, v1b, b1, v2, b2,
                   v3, b3, g, be, ws_lo, ws_hi, wd_lo, wd_hi, *outs):
    agg = a0_r[...] + a1_r[...]
    h = jnp.maximum(_mm(node_r[...], v1a[...]) + _mm(agg, v1b[...])
                    + b1[...], 0.0)
    h = jnp.maximum(_mm(h, v2[...]) + b2[...], 0.0)
    h = _mm(h, v3[...]) + b3[...]
    node = node_r[...] + _ln(h, g[...], be[...])
    outs[0][...] = node
    if with_proj:
        outs[1][...] = _proj_pack(node, ws_lo, ws_hi)
        outs[2][...] = _proj_pack(node, wd_lo, wd_hi)


def _knn_body(n_coarse, pf_r, pcT_r, node_r, xc_r,
              pw1, pb1, pw2, pb2, pw3, pb3, pg, pbe,
              dw1, db1, dw2, db2, dw3, db3, out_o):
    bf = node_r.shape[0]
    cp = pcT_r.shape[1]
    inf = jnp.float32(jnp.inf)
    d2 = jnp.zeros((bf, cp), F32)
    for d in range(3):
        diff = pf_r[:, d:d + 1] - pcT_r[d, :][None, :]
        d2 = d2 + diff * diff
    colj = lax.broadcasted_iota(jnp.int32, (bf, cp), 1)
    d2 = jnp.where(colj >= n_coarse, inf, d2)
    wacc = jnp.zeros((bf, cp), F32)
    wsum = jnp.zeros((bf, 1), F32)
    for _k in range(3):
        m = jnp.min(d2, axis=1, keepdims=True)
        idx = jnp.min(jnp.where(d2 == m, colj, cp), axis=1, keepdims=True)
        sel = colj == idx
        w = 1.0 / jnp.maximum(m, 1e-16)
        wacc = wacc + jnp.where(sel, w, 0.0)
        wsum = wsum + w
        d2 = jnp.where(sel, inf, d2)
    interp = _mm(wacc, xc_r[...]) / wsum
    nf = node_r[...] + interp
    h = jnp.maximum(_mm(nf, pw1[...]) + pb1[...], 0.0)
    h = jnp.maximum(_mm(h, pw2[...]) + pb2[...], 0.0)
    h = _mm(h, pw3[...]) + pb3[...]
    h = _ln(h, pg[...], pbe[...])
    h = jnp.maximum(_mm(h, dw1[...]) + db1[...], 0.0)
    h = jnp.maximum(_mm(h, dw2[...]) + db2[...], 0.0)
    out_o[...] = _mm(h, dw3[...]) + db3[...]


def _row_spec(b, d):
    return pl.BlockSpec((b, d), lambda i: (i, 0))


def _full_spec(shape):
    return pl.BlockSpec(shape, lambda i: tuple(0 for _ in shape))


def _wspecs(ws):
    return [_full_spec(w.shape) for w in ws]


def _enc_node_call(x, ws, n):
    f = pl.pallas_call(
        _enc_node_body,
        grid=(n // BN,),
        in_specs=[_row_spec(BN, x.shape[1])] + _wspecs(ws),
        out_specs=[_row_spec(BN, 128), _row_spec(BN, 64), _row_spec(BN, 64)],
        out_shape=[jax.ShapeDtypeStruct((n, 128), F32),
                   jax.ShapeDtypeStruct((n, 64), F32),
                   jax.ShapeDtypeStruct((n, 64), F32)],
    )
    return f(x, *ws)


def _enc_edge_call(ea, ws, e, off_blocks):
    f = pl.pallas_call(
        _enc_edge_body,
        grid=(e // BE,),
        in_specs=[pl.BlockSpec((BE, ea.shape[1]),
                               lambda i: (i + off_blocks, 0))] + _wspecs(ws),
        out_specs=_row_spec(BE, 128),
        out_shape=jax.ShapeDtypeStruct((e, 128), BF16),
    )
    return f(ea, *ws)


def _edge_mlp_call(edge, gv, ws, e, with_new):
    nblk = e // BE
    n_out = 2 if with_new else 1
    f = pl.pallas_call(
        functools.partial(_edge_mlp_body, with_new),
        grid=(nblk,),
        in_specs=[_row_spec(BE, 128), _row_spec(BE, 128)] + _wspecs(ws),
        out_specs=[_row_spec(BE, 128)] * n_out,
        out_shape=[jax.ShapeDtypeStruct((e, 128), F32),
                   jax.ShapeDtypeStruct((e, 128), BF16)][:n_out],
    )
    return f(edge, gv, *ws)


def _node_mlp_call(node, a0, a1, ws, n, with_proj):
    nblk = n // BN
    n_out = 3 if with_proj else 1
    f = pl.pallas_call(
        functools.partial(_node_mlp_body, with_proj),
        grid=(nblk,),
        in_specs=[_row_spec(BN, 128)] * 3 + _wspecs(ws),
        out_specs=[_row_spec(BN, 128)] + [_row_spec(BN, 64)] * (n_out - 1),
        out_shape=[jax.ShapeDtypeStruct((n, 128), F32)] +
                  [jax.ShapeDtypeStruct((n, 64), F32)] * (n_out - 1),
    )
    return f(node, a0, a1, *ws)


def _knn_call(posf, pcT, node, xc, ws, n, n_coarse):
    cp = pcT.shape[1]
    f = pl.pallas_call(
        functools.partial(_knn_body, n_coarse),
        grid=(n // BF,),
        in_specs=[_row_spec(BF, 8), _full_spec((8, cp)),
                  _row_spec(BF, 128), _full_spec((cp, 128))] + _wspecs(ws),
        out_specs=_row_spec(BF, 128),
        out_shape=jax.ShapeDtypeStruct((n, 128), F32),
    )
    return f(posf, pcT, node, xc, *ws)


def _sc_gather(table, idx, chunk):
    b = idx.shape[0]
    d = table.shape[1]
    bpw = b // NWORK
    nchunks = bpw // chunk
    assert bpw % chunk == 0 and chunk % 16 == 0 and b % NWORK == 0
    mesh = plsc.VectorSubcoreMesh(core_axis_name="c", subcore_axis_name="s")

    @functools.partial(
        pl.kernel,
        out_type=jax.ShapeDtypeStruct((b, d), table.dtype),
        mesh=mesh,
        compiler_params=pltpu.CompilerParams(use_tc_tiling_on_sc=False),
        scratch_types=[
            pltpu.VMEM((chunk,), jnp.int32),
            pltpu.VMEM((chunk, d), table.dtype),
            pltpu.SemaphoreType.DMA,
        ],
    )
    def k(table_h, idx_h, out_h, idx_v, rows_v, sem):
        wid = lax.axis_index("s") * NCORES + lax.axis_index("c")
        base0 = wid * bpw

        def body(c, carry):
            base = base0 + c * chunk
            pltpu.sync_copy(idx_h.at[pl.ds(base, chunk)], idx_v)
            pltpu.async_copy(table_h.at[idx_v], rows_v, sem).wait()
            pltpu.sync_copy(rows_v, out_h.at[pl.ds(base, chunk)])
            return carry

        lax.fori_loop(0, nchunks, body, 0)

    return k(table, idx)


def _coop_rows(sid, rows, fn):
    q = (-(-rows // NSUB) + 7) // 8 * 8
    q_last = rows - (NSUB - 1) * q
    assert 0 < q_last <= q and q_last % 8 == 0

    @pl.when(sid < NSUB - 1)
    def _():
        fn(sid * q, q)

    @pl.when(sid == NSUB - 1)
    def _():
        fn((NSUB - 1) * q, q_last)


def _sc_segsum(vals, idx, n_out, chunk):
    e = idx.shape[0]
    d = vals.shape[1]
    half = (n_out // 2 + 7) // 8 * 8
    ept = e // NSUB
    nchunks = ept // chunk
    nv = chunk // 16
    assert ept % chunk == 0 and chunk % 16 == 0 and half % 8 == 0
    zeros = jnp.zeros((half + 8, d), F32)
    mesh = plsc.VectorSubcoreMesh(core_axis_name="c", subcore_axis_name="s")

    @functools.partial(
        pl.kernel,
        out_type=jax.ShapeDtypeStruct((n_out, d), F32),
        mesh=mesh,
        scratch_types=[
            pltpu.VMEM((chunk,), jnp.int32),
            pltpu.VMEM((chunk, d), F32),
            pltpu.VMEM_SHARED((half + 8, d), F32),
        ],
    )
    def k(vals_h, idx_h, zer_h, out_h, idx_v, rows_v, shared):
        cid = lax.axis_index("c")
        sid = lax.axis_index("s")
        lo = cid * half
        hi = jnp.where(cid == 0, half, n_out - half)

        _coop_rows(sid, half + 8,
                   lambda o, s: pltpu.sync_copy(zer_h.at[pl.ds(o, s)],
                                                shared.at[pl.ds(o, s)]))
        plsc.subcore_barrier()

        def body(c, carry):
            base = sid * ept + c * chunk
            pltpu.sync_copy(idx_h.at[pl.ds(base, chunk)], idx_v)
            pltpu.sync_copy(vals_h.at[pl.ds(base, chunk)], rows_v)

            def remap(j, carry2):
                v = idx_v[pl.ds(j * 16, 16)] - lo
                oob = (v < 0) | (v >= hi)
                idx_v[pl.ds(j * 16, 16)] = jnp.where(oob, half, v)
                return carry2

            lax.fori_loop(0, nv, remap, 0)
            pltpu.sync_copy(rows_v, shared.at[idx_v], add=True)
            return carry

        lax.fori_loop(0, nchunks, body, 0)
        plsc.subcore_barrier()

        @pl.when(cid == 0)
        def _():
            _coop_rows(sid, half,
                       lambda o, s: pltpu.sync_copy(
                           shared.at[pl.ds(o, s)], out_h.at[pl.ds(o, s)]))

        @pl.when(cid == 1)
        def _():
            _coop_rows(sid, n_out - half,
                       lambda o, s: pltpu.sync_copy(
                           shared.at[pl.ds(o, s)],
                           out_h.at[pl.ds(half + o, s)]))

    return k(vals, idx, zeros)


def _mlp_ws(p, pad_last_to=None):
    (w1, b1), (w2, b2), (w3, b3) = p["layers"]
    g, be = p["ln"]
    return [w1, b1.reshape(1, -1), w2, b2.reshape(1, -1), w3,
            b3.reshape(1, -1), g.reshape(1, -1), be.reshape(1, -1)]


def kernel(x, edge_index, edge_attr, pos, coarse_idx, params):
    n = x.shape[0]
    e = edge_attr.shape[0]
    n_coarse = coarse_idx.shape[0]
    src = edge_index[0].astype(jnp.int32)
    dst = edge_index[1].astype(jnp.int32)

    pe = params["proc_edge"]
    pe_w1 = pe["layers"][0][0]
    w1a, w1b, w1c = pe_w1[:128], pe_w1[128:256], pe_w1[256:384]
    proj_ws = [w1b[:, :64], w1b[:, 64:], w1c[:, :64], w1c[:, 64:]]
    i64 = jnp.eye(64, dtype=BF16)
    z64 = jnp.zeros((128, 64), BF16)
    ii = jnp.concatenate([i64, i64], axis=0)
    m1 = jnp.concatenate([ii, z64], axis=1)
    m2 = jnp.concatenate([z64, ii], axis=1)
    pe_ws = [w1a.astype(BF16), m1, m2, pe["layers"][0][1].reshape(1, -1),
             pe["layers"][1][0].astype(BF16),
             pe["layers"][1][1].reshape(1, -1),
             pe["layers"][2][0].astype(BF16),
             pe["layers"][2][1].reshape(1, -1),
             pe["ln"][0].reshape(1, -1), pe["ln"][1].reshape(1, -1)]

    pn = params["proc_node"]
    pn_w1 = pn["layers"][0][0]
    v1a, v1b = pn_w1[:128], pn_w1[128:]
    pn_ws = [v1a, v1b, pn["layers"][0][1].reshape(1, -1),
             pn["layers"][1][0], pn["layers"][1][1].reshape(1, -1),
             pn["layers"][2][0], pn["layers"][2][1].reshape(1, -1),
             pn["ln"][0].reshape(1, -1), pn["ln"][1].reshape(1, -1),
             ] + proj_ws

    node, ps, pd_ = _enc_node_call(x, _mlp_ws(params["enc_node"]) + proj_ws, n)
    ee_ws = _mlp_ws(params["enc_edge"])
    for i in (0, 2, 4):
        ee_ws[i] = ee_ws[i].astype(BF16)

    e2 = e // 2
    halves = []
    for h in range(2):
        sl = slice(h * e2, (h + 1) * e2)
        halves.append({
            "idx_cat": jnp.stack([src[sl], dst[sl] + n], axis=1).reshape(-1),
            "dst": dst[sl],
            "edge": _enc_edge_call(edge_attr, ee_ws, e2, h * (e2 // BE)),
        })

    for step in range(2):
        last = step == 1
        p_cat = jnp.concatenate([ps, pd_], axis=0)
        gc, upd, agg = [None, None], [None, None], [None, None]
        for h in range(2):
            gc[h] = _sc_gather(p_cat, halves[h]["idx_cat"], 400
                               ).reshape(e2, 128)
        for h in range(2):
            if last:
                (upd[h],) = _edge_mlp_call(halves[h]["edge"], gc[h], pe_ws,
                                           e2, with_new=False)
            else:
                upd[h], halves[h]["edge"] = _edge_mlp_call(
                    halves[h]["edge"], gc[h], pe_ws, e2, with_new=True)
            agg[h] = _sc_segsum(upd[h], halves[h]["dst"], n, 400)
        if last:
            (node,) = _node_mlp_call(node, agg[0], agg[1], pn_ws, n,
                                     with_proj=False)
        else:
            node, ps, pd_ = _node_mlp_call(node, agg[0], agg[1], pn_ws, n,
                                           with_proj=True)

    cpad = ((n_coarse + 8 * NWORK - 1) // (8 * NWORK)) * (8 * NWORK)
    cidx = jnp.concatenate(
        [coarse_idx.astype(jnp.int32),
         jnp.zeros((cpad - n_coarse,), jnp.int32)])
    xc = _sc_gather(node, cidx, cpad // NWORK)

    pos_c = jnp.take(pos, coarse_idx, axis=0)
    pcT = jnp.zeros((8, cpad), F32).at[:3, :n_coarse].set(pos_c.T)
    posf = jnp.pad(pos, ((0, 0), (0, 5)))

    po_ws = _mlp_ws(params["post"])
    (dw1, db1), (dw2, db2), (dw3, db3) = params["dec"]["layers"]
    out_dim = dw3.shape[1]
    dw3p = jnp.zeros((128, 128), F32).at[:, :out_dim].set(dw3)
    db3p = jnp.zeros((1, 128), F32).at[0, :out_dim].set(db3)
    de_ws = [dw1, db1.reshape(1, -1), dw2, db2.reshape(1, -1), dw3p, db3p]

    outp = _knn_call(posf, pcT, node, xc, po_ws + de_ws, n, n_coarse)
    return outp[:, :out_dim]


# speedup vs baseline: 1.1176x; 1.1176x over previous
import functools

import jax
import jax.numpy as jnp
from jax import lax
from jax.experimental import pallas as pl
from jax.experimental.pallas import tpu as pltpu
from jax.experimental.pallas import tpu_sc as plsc

F32 = jnp.float32
BF16 = jnp.bfloat16
EPS = 1e-5
NCORES, NSUB = 2, 16
NWORK = NCORES * NSUB

BN = 2000
BE = 2000
BF = 400


def _pack2(pa, pb):
    ua = lax.bitcast_convert_type(pa, jnp.uint32)
    ub = lax.bitcast_convert_type(pb, jnp.uint32)
    ra = (ua + ((ua >> 16) & 1) + 0x7FFF) >> 16
    rb = (ub + ((ub >> 16) & 1) + 0x7FFF) & jnp.uint32(0xFFFF0000)
    return lax.bitcast_convert_type(ra | rb, F32)


def _unpack2(g):
    u = lax.bitcast_convert_type(g, jnp.uint32)
    lo = lax.bitcast_convert_type(u << 16, F32)
    hi = lax.bitcast_convert_type(u & jnp.uint32(0xFFFF0000), F32)
    return lo, hi


def _ln(h, g, be):
    m = jnp.mean(h, axis=-1, keepdims=True)
    v = jnp.mean((h - m) ** 2, axis=-1, keepdims=True)
    return (h - m) * lax.rsqrt(v + EPS) * g + be


def _mm(a, b):
    return jnp.dot(a, b, preferred_element_type=F32)


def _proj_pack(node, ws_lo, ws_hi):
    return _pack2(_mm(node, ws_lo[...]), _mm(node, ws_hi[...]))


def _enc_node_body(x_r, w1, b1, w2, b2, w3, b3, g, be,
                   ws_lo, ws_hi, wd_lo, wd_hi, node_o, ps_o, pd_o):
    h = jnp.maximum(_mm(x_r[...], w1[...]) + b1[...], 0.0)
    h = jnp.maximum(_mm(h, w2[...]) + b2[...], 0.0)
    h = _mm(h, w3[...]) + b3[...]
    node = _ln(h, g[...], be[...])
    node_o[...] = node
    ps_o[...] = _proj_pack(node, ws_lo, ws_hi)
    pd_o[...] = _proj_pack(node, wd_lo, wd_hi)


def _enc_edge_body(x_r, w1, b1, w2, b2, w3, b3, g, be, edge_o):
    h = jnp.maximum(_mm(x_r[...].astype(BF16), w1[...]) + b1[...], 0.0)
    h = jnp.maximum(_mm(h.astype(BF16), w2[...]) + b2[...], 0.0)
    h = _mm(h.astype(BF16), w3[...]) + b3[...]
    edge_o[...] = _ln(h, g[...], be[...]).astype(BF16)


def _edge_mlp_body(with_new, edge_r, g_r, w1a, m1, m2, b1, w2, b2, w3, b3,
                   g, be, *outs):
    u = lax.bitcast_convert_type(g_r[...], jnp.uint32)
    lo = lax.bitcast_convert_type(u << 16, F32).astype(BF16)
    hi = lax.bitcast_convert_type(u & jnp.uint32(0xFFFF0000), F32).astype(BF16)
    h1 = jnp.maximum(_mm(edge_r[...], w1a[...]) + _mm(lo, m1[...])
                     + _mm(hi, m2[...]) + b1[...], 0.0)
    h2 = jnp.maximum(_mm(h1.astype(BF16), w2[...]) + b2[...], 0.0)
    h3 = _mm(h2.astype(BF16), w3[...]) + b3[...]
    upd = _ln(h3, g[...], be[...])
    outs[0][...] = upd
    if with_new:
        outs[1][...] = (edge_r[...].astype(F32) + upd).astype(BF16)


def _node_mlp_body(with_proj, node_r, a0_r, a1_r, v1a, v1b, b1, v2, b2,
                   v3, b3, g, be, ws_lo, ws_hi, wd_lo, wd_hi, *outs):
    agg = a0_r[...] + a1_r[...]
    h = jnp.maximum(_mm(node_r[...], v1a[...]) + _mm(agg, v1b[...])
                    + b1[...], 0.0)
    h = jnp.maximum(_mm(h, v2[...]) + b2[...], 0.0)
    h = _mm(h, v3[...]) + b3[...]
    node = node_r[...] + _ln(h, g[...], be[...])
    outs[0][...] = node
    if with_proj:
        outs[1][...] = _proj_pack(node, ws_lo, ws_hi)
        outs[2][...] = _proj_pack(node, wd_lo, wd_hi)


def _knn_body(n_coarse, pf_r, pcT_r, node_r, xc_r,
              pw1, pb1, pw2, pb2, pw3, pb3, pg, pbe,
              dw1, db1, dw2, db2, dw3, db3, out_o):
    bf = node_r.shape[0]
    cp = pcT_r.shape[1]
    inf = jnp.float32(jnp.inf)
    d2 = jnp.zeros((bf, cp), F32)
    for d in range(3):
        diff = pf_r[:, d:d + 1] - pcT_r[d, :][None, :]
        d2 = d2 + diff * diff
    colj = lax.broadcasted_iota(jnp.int32, (bf, cp), 1)
    d2 = jnp.where(colj >= n_coarse, inf, d2)
    wacc = jnp.zeros((bf, cp), F32)
    wsum = jnp.zeros((bf, 1), F32)
    for _k in range(3):
        m = jnp.min(d2, axis=1, keepdims=True)
        idx = jnp.min(jnp.where(d2 == m, colj, cp), axis=1, keepdims=True)
        sel = colj == idx
        w = 1.0 / jnp.maximum(m, 1e-16)
        wacc = wacc + jnp.where(sel, w, 0.0)
        wsum = wsum + w
        d2 = jnp.where(sel, inf, d2)
    interp = _mm(wacc, xc_r[...]) / wsum
    nf = node_r[...] + interp
    h = jnp.maximum(_mm(nf, pw1[...]) + pb1[...], 0.0)
    h = jnp.maximum(_mm(h, pw2[...]) + pb2[...], 0.0)
    h = _mm(h, pw3[...]) + pb3[...]
    h = _ln(h, pg[...], pbe[...])
    h = jnp.maximum(_mm(h, dw1[...]) + db1[...], 0.0)
    h = jnp.maximum(_mm(h, dw2[...]) + db2[...], 0.0)
    out_o[...] = _mm(h, dw3[...]) + db3[...]


def _row_spec(b, d):
    return pl.BlockSpec((b, d), lambda i: (i, 0))


def _full_spec(shape):
    return pl.BlockSpec(shape, lambda i: tuple(0 for _ in shape))


def _wspecs(ws):
    return [_full_spec(w.shape) for w in ws]


def _enc_node_call(x, ws, n):
    f = pl.pallas_call(
        _enc_node_body,
        grid=(n // BN,),
        in_specs=[_row_spec(BN, x.shape[1])] + _wspecs(ws),
        out_specs=[_row_spec(BN, 128), _row_spec(BN, 64), _row_spec(BN, 64)],
        out_shape=[jax.ShapeDtypeStruct((n, 128), F32),
                   jax.ShapeDtypeStruct((n, 64), F32),
                   jax.ShapeDtypeStruct((n, 64), F32)],
    )
    return f(x, *ws)


def _enc_edge_call(ea, ws, e, off_blocks):
    f = pl.pallas_call(
        _enc_edge_body,
        grid=(e // BE,),
        in_specs=[pl.BlockSpec((BE, ea.shape[1]),
                               lambda i: (i + off_blocks, 0))] + _wspecs(ws),
        out_specs=_row_spec(BE, 128),
        out_shape=jax.ShapeDtypeStruct((e, 128), BF16),
    )
    return f(ea, *ws)


def _edge_mlp_call(edge, gv, ws, e, with_new):
    nblk = e // BE
    n_out = 2 if with_new else 1
    f = pl.pallas_call(
        functools.partial(_edge_mlp_body, with_new),
        grid=(nblk,),
        in_specs=[_row_spec(BE, 128), _row_spec(BE, 128)] + _wspecs(ws),
        out_specs=[_row_spec(BE, 128)] * n_out,
        out_shape=[jax.ShapeDtypeStruct((e, 128), F32),
                   jax.ShapeDtypeStruct((e, 128), BF16)][:n_out],
    )
    return f(edge, gv, *ws)


def _node_mlp_call(node, a0, a1, ws, n, with_proj):
    nblk = n // BN
    n_out = 3 if with_proj else 1
    f = pl.pallas_call(
        functools.partial(_node_mlp_body, with_proj),
        grid=(nblk,),
        in_specs=[_row_spec(BN, 128)] * 3 + _wspecs(ws),
        out_specs=[_row_spec(BN, 128)] + [_row_spec(BN, 64)] * (n_out - 1),
        out_shape=[jax.ShapeDtypeStruct((n, 128), F32)] +
                  [jax.ShapeDtypeStruct((n, 64), F32)] * (n_out - 1),
    )
    return f(node, a0, a1, *ws)


def _knn_call(posf, pcT, node, xc, ws, n, n_coarse):
    cp = pcT.shape[1]
    f = pl.pallas_call(
        functools.partial(_knn_body, n_coarse),
        grid=(n // BF,),
        in_specs=[_row_spec(BF, 8), _full_spec((8, cp)),
                  _row_spec(BF, 128), _full_spec((cp, 128))] + _wspecs(ws),
        out_specs=_row_spec(BF, 128),
        out_shape=jax.ShapeDtypeStruct((n, 128), F32),
    )
    return f(posf, pcT, node, xc, *ws)


def _sc_gather(table, idx, chunk):
    b = idx.shape[0]
    d = table.shape[1]
    bpw = b // NWORK
    nchunks = bpw // chunk
    assert bpw % chunk == 0 and chunk % 16 == 0 and b % NWORK == 0
    mesh = plsc.VectorSubcoreMesh(core_axis_name="c", subcore_axis_name="s")

    @functools.partial(
        pl.kernel,
        out_type=jax.ShapeDtypeStruct((b, d), table.dtype),
        mesh=mesh,
        compiler_params=pltpu.CompilerParams(use_tc_tiling_on_sc=False),
        scratch_types=[
            pltpu.VMEM((chunk,), jnp.int32),
            pltpu.VMEM((chunk, d), table.dtype),
            pltpu.SemaphoreType.DMA,
        ],
    )
    def k(table_h, idx_h, out_h, idx_v, rows_v, sem):
        wid = lax.axis_index("s") * NCORES + lax.axis_index("c")
        base0 = wid * bpw

        def body(c, carry):
            base = base0 + c * chunk
            pltpu.sync_copy(idx_h.at[pl.ds(base, chunk)], idx_v)
            pltpu.async_copy(table_h.at[idx_v], rows_v, sem).wait()
            pltpu.sync_copy(rows_v, out_h.at[pl.ds(base, chunk)])
            return carry

        lax.fori_loop(0, nchunks, body, 0)

    return k(table, idx)


def _coop_rows(sid, rows, fn):
    q = (-(-rows // NSUB) + 7) // 8 * 8
    q_last = rows - (NSUB - 1) * q
    assert 0 < q_last <= q and q_last % 8 == 0

    @pl.when(sid < NSUB - 1)
    def _():
        fn(sid * q, q)

    @pl.when(sid == NSUB - 1)
    def _():
        fn((NSUB - 1) * q, q_last)


def _sc_segsum(vals, idx, n_out, chunk):
    e = idx.shape[0]
    d = vals.shape[1]
    dh = d // 2
    ept = e // NSUB
    nchunks = ept // chunk
    assert ept % chunk == 0 and chunk % 8 == 0
    zeros = jnp.zeros((n_out, dh), F32)
    mesh = plsc.VectorSubcoreMesh(core_axis_name="c", subcore_axis_name="s")

    @functools.partial(
        pl.kernel,
        out_type=jax.ShapeDtypeStruct((n_out, d), F32),
        mesh=mesh,
        compiler_params=pltpu.CompilerParams(use_tc_tiling_on_sc=False),
        scratch_types=[
            pltpu.VMEM((chunk,), jnp.int32),
            pltpu.VMEM((chunk, dh), F32),
            pltpu.VMEM_SHARED((n_out, dh), F32),
        ],
    )
    def k(vals_h, idx_h, zer_h, out_h, idx_v, rows_v, shared):
        cid = lax.axis_index("c")
        sid = lax.axis_index("s")

        _coop_rows(sid, n_out,
                   lambda o, s: pltpu.sync_copy(zer_h.at[pl.ds(o, s)],
                                                shared.at[pl.ds(o, s)]))
        plsc.subcore_barrier()

        def run(c0):
            def body(c, carry):
                base = sid * ept + c * chunk
                pltpu.sync_copy(idx_h.at[pl.ds(base, chunk)], idx_v)
                pltpu.sync_copy(
                    vals_h.at[pl.ds(base, chunk), pl.ds(c0, dh)], rows_v)
                pltpu.sync_copy(rows_v, shared.at[idx_v], add=True)
                return carry

            lax.fori_loop(0, nchunks, body, 0)

        def flush(c0):
            _coop_rows(sid, n_out,
                       lambda o, s: pltpu.sync_copy(
                           shared.at[pl.ds(o, s)],
                           out_h.at[pl.ds(o, s), pl.ds(c0, dh)]))

        @pl.when(cid == 0)
        def _():
            run(0)

        @pl.when(cid == 1)
        def _():
            run(dh)

        plsc.subcore_barrier()

        @pl.when(cid == 0)
        def _():
            flush(0)

        @pl.when(cid == 1)
        def _():
            flush(dh)

    return k(vals, idx, zeros)


def _mlp_ws(p, pad_last_to=None):
    (w1, b1), (w2, b2), (w3, b3) = p["layers"]
    g, be = p["ln"]
    return [w1, b1.reshape(1, -1), w2, b2.reshape(1, -1), w3,
            b3.reshape(1, -1), g.reshape(1, -1), be.reshape(1, -1)]


def kernel(x, edge_index, edge_attr, pos, coarse_idx, params):
    n = x.shape[0]
    e = edge_attr.shape[0]
    n_coarse = coarse_idx.shape[0]
    src = edge_index[0].astype(jnp.int32)
    dst = edge_index[1].astype(jnp.int32)

    pe = params["proc_edge"]
    pe_w1 = pe["layers"][0][0]
    w1a, w1b, w1c = pe_w1[:128], pe_w1[128:256], pe_w1[256:384]
    proj_ws = [w1b[:, :64], w1b[:, 64:], w1c[:, :64], w1c[:, 64:]]
    i64 = jnp.eye(64, dtype=BF16)
    z64 = jnp.zeros((128, 64), BF16)
    ii = jnp.concatenate([i64, i64], axis=0)
    m1 = jnp.concatenate([ii, z64], axis=1)
    m2 = jnp.concatenate([z64, ii], axis=1)
    pe_ws = [w1a.astype(BF16), m1, m2, pe["layers"][0][1].reshape(1, -1),
             pe["layers"][1][0].astype(BF16),
             pe["layers"][1][1].reshape(1, -1),
             pe["layers"][2][0].astype(BF16),
             pe["layers"][2][1].reshape(1, -1),
             pe["ln"][0].reshape(1, -1), pe["ln"][1].reshape(1, -1)]

    pn = params["proc_node"]
    pn_w1 = pn["layers"][0][0]
    v1a, v1b = pn_w1[:128], pn_w1[128:]
    pn_ws = [v1a, v1b, pn["layers"][0][1].reshape(1, -1),
             pn["layers"][1][0], pn["layers"][1][1].reshape(1, -1),
             pn["layers"][2][0], pn["layers"][2][1].reshape(1, -1),
             pn["ln"][0].reshape(1, -1), pn["ln"][1].reshape(1, -1),
             ] + proj_ws

    node, ps, pd_ = _enc_node_call(x, _mlp_ws(params["enc_node"]) + proj_ws, n)
    ee_ws = _mlp_ws(params["enc_edge"])
    for i in (0, 2, 4):
        ee_ws[i] = ee_ws[i].astype(BF16)

    e2 = e // 2
    halves = []
    for h in range(2):
        sl = slice(h * e2, (h + 1) * e2)
        halves.append({
            "idx_cat": jnp.stack([src[sl], dst[sl] + n], axis=1).reshape(-1),
            "dst": dst[sl],
            "edge": _enc_edge_call(edge_attr, ee_ws, e2, h * (e2 // BE)),
        })

    for step in range(2):
        last = step == 1
        p_cat = jnp.concatenate([ps, pd_], axis=0)
        gc, upd, agg = [None, None], [None, None], [None, None]
        for h in range(2):
            gc[h] = _sc_gather(p_cat, halves[h]["idx_cat"], 400
                               ).reshape(e2, 128)
        for h in range(2):
            if last:
                (upd[h],) = _edge_mlp_call(halves[h]["edge"], gc[h], pe_ws,
                                           e2, with_new=False)
            else:
                upd[h], halves[h]["edge"] = _edge_mlp_call(
                    halves[h]["edge"], gc[h], pe_ws, e2, with_new=True)
            agg[h] = _sc_segsum(upd[h], halves[h]["dst"], n, 1000)
        if last:
            (node,) = _node_mlp_call(node, agg[0], agg[1], pn_ws, n,
                                     with_proj=False)
        else:
            node, ps, pd_ = _node_mlp_call(node, agg[0], agg[1], pn_ws, n,
                                           with_proj=True)

    cpad = ((n_coarse + 8 * NWORK - 1) // (8 * NWORK)) * (8 * NWORK)
    cidx = jnp.concatenate(
        [coarse_idx.astype(jnp.int32),
         jnp.zeros((cpad - n_coarse,), jnp.int32)])
    xc = _sc_gather(node, cidx, cpad // NWORK)

    pos_c = jnp.take(pos, coarse_idx, axis=0)
    pcT = jnp.zeros((8, cpad), F32).at[:3, :n_coarse].set(pos_c.T)
    posf = jnp.pad(pos, ((0, 0), (0, 5)))

    po_ws = _mlp_ws(params["post"])
    (dw1, db1), (dw2, db2), (dw3, db3) = params["dec"]["layers"]
    out_dim = dw3.shape[1]
    dw3p = jnp.zeros((128, 128), F32).at[:, :out_dim].set(dw3)
    db3p = jnp.zeros((1, 128), F32).at[0, :out_dim].set(db3)
    de_ws = [dw1, db1.reshape(1, -1), dw2, db2.reshape(1, -1), dw3p, db3p]

    outp = _knn_call(posf, pcT, node, xc, po_ws + de_ws, n, n_coarse)
    return outp[:, :out_dim]


# speedup vs baseline: 1.2762x; 1.1419x over previous
import functools

import jax
import jax.numpy as jnp
from jax import lax
from jax.experimental import pallas as pl
from jax.experimental.pallas import tpu as pltpu
from jax.experimental.pallas import tpu_sc as plsc

F32 = jnp.float32
BF16 = jnp.bfloat16
EPS = 1e-5
NCORES, NSUB = 2, 16
NWORK = NCORES * NSUB

BN = 2000
BE = 4000
BF = 400


def _pack2(pa, pb):
    ua = lax.bitcast_convert_type(pa, jnp.uint32)
    ub = lax.bitcast_convert_type(pb, jnp.uint32)
    ra = (ua + ((ua >> 16) & 1) + 0x7FFF) >> 16
    rb = (ub + ((ub >> 16) & 1) + 0x7FFF) & jnp.uint32(0xFFFF0000)
    return lax.bitcast_convert_type(ra | rb, F32)


def _unpack2(g):
    u = lax.bitcast_convert_type(g, jnp.uint32)
    lo = lax.bitcast_convert_type(u << 16, F32)
    hi = lax.bitcast_convert_type(u & jnp.uint32(0xFFFF0000), F32)
    return lo, hi


def _ln(h, g, be):
    m = jnp.mean(h, axis=-1, keepdims=True)
    v = jnp.mean((h - m) ** 2, axis=-1, keepdims=True)
    return (h - m) * lax.rsqrt(v + EPS) * g + be


def _mm(a, b):
    return jnp.dot(a, b, preferred_element_type=F32)


def _proj_pack(node, ws_lo, ws_hi):
    return _pack2(_mm(node, ws_lo[...]), _mm(node, ws_hi[...]))


def _enc_node_body(x_r, w1, b1, w2, b2, w3, b3, g, be,
                   ws_lo, ws_hi, wd_lo, wd_hi, node_o, ps_o, pd_o):
    h = jnp.maximum(_mm(x_r[...], w1[...]) + b1[...], 0.0)
    h = jnp.maximum(_mm(h, w2[...]) + b2[...], 0.0)
    h = _mm(h, w3[...]) + b3[...]
    node = _ln(h, g[...], be[...])
    node_o[...] = node
    ps_o[...] = _proj_pack(node, ws_lo, ws_hi)
    pd_o[...] = _proj_pack(node, wd_lo, wd_hi)


def _enc_edge_body(x_r, w1, b1, w2, b2, w3, b3, g, be, edge_o):
    h = jnp.maximum(_mm(x_r[...].astype(BF16), w1[...]) + b1[...], 0.0)
    h = jnp.maximum(_mm(h.astype(BF16), w2[...]) + b2[...], 0.0)
    h = _mm(h.astype(BF16), w3[...]) + b3[...]
    edge_o[...] = _ln(h, g[...], be[...]).astype(BF16)


def _edge_mlp_body(with_new, edge_r, g_r, w1a, m1, m2, b1, w2, b2, w3, b3,
                   g, be, *outs):
    u = lax.bitcast_convert_type(g_r[...], jnp.uint32)
    lo = lax.bitcast_convert_type(u << 16, F32).astype(BF16)
    hi = lax.bitcast_convert_type(u & jnp.uint32(0xFFFF0000), F32).astype(BF16)
    h1 = jnp.maximum(_mm(edge_r[...], w1a[...]) + _mm(lo, m1[...])
                     + _mm(hi, m2[...]) + b1[...], 0.0)
    h2 = jnp.maximum(_mm(h1.astype(BF16), w2[...]) + b2[...], 0.0)
    h3 = _mm(h2.astype(BF16), w3[...]) + b3[...]
    upd = _ln(h3, g[...], be[...])
    outs[0][...] = upd
    if with_new:
        outs[1][...] = (edge_r[...].astype(F32) + upd).astype(BF16)


def _node_mlp_body(with_proj, node_r, a0_r, a1_r, v1a, v1b, b1, v2, b2,
                   v3, b3, g, be, ws_lo, ws_hi, wd_lo, wd_hi, *outs):
    agg = a0_r[...] + a1_r[...]
    h = jnp.maximum(_mm(node_r[...], v1a[...]) + _mm(agg, v1b[...])
                    + b1[...], 0.0)
    h = jnp.maximum(_mm(h, v2[...]) + b2[...], 0.0)
    h = _mm(h, v3[...]) + b3[...]
    node = node_r[...] + _ln(h, g[...], be[...])
    outs[0][...] = node
    if with_proj:
        outs[1][...] = _proj_pack(node, ws_lo, ws_hi)
        outs[2][...] = _proj_pack(node, wd_lo, wd_hi)


def _knn_body(n_coarse, pf_r, pcT_r, node_r, xc_r,
              pw1, pb1, pw2, pb2, pw3, pb3, pg, pbe,
              dw1, db1, dw2, db2, dw3, db3, out_o):
    bf = node_r.shape[0]
    cp = pcT_r.shape[1]
    inf = jnp.float32(jnp.inf)
    d2 = jnp.zeros((bf, cp), F32)
    for d in range(3):
        diff = pf_r[:, d:d + 1] - pcT_r[d, :][None, :]
        d2 = d2 + diff * diff
    colj = lax.broadcasted_iota(jnp.int32, (bf, cp), 1)
    d2 = jnp.where(colj >= n_coarse, inf, d2)
    wacc = jnp.zeros((bf, cp), F32)
    wsum = jnp.zeros((bf, 1), F32)
    for _k in range(3):
        m = jnp.min(d2, axis=1, keepdims=True)
        idx = jnp.min(jnp.where(d2 == m, colj, cp), axis=1, keepdims=True)
        sel = colj == idx
        w = 1.0 / jnp.maximum(m, 1e-16)
        wacc = wacc + jnp.where(sel, w, 0.0)
        wsum = wsum + w
        d2 = jnp.where(sel, inf, d2)
    interp = _mm(wacc, xc_r[...]) / wsum
    nf = node_r[...] + interp
    h = jnp.maximum(_mm(nf, pw1[...]) + pb1[...], 0.0)
    h = jnp.maximum(_mm(h, pw2[...]) + pb2[...], 0.0)
    h = _mm(h, pw3[...]) + pb3[...]
    h = _ln(h, pg[...], pbe[...])
    h = jnp.maximum(_mm(h, dw1[...]) + db1[...], 0.0)
    h = jnp.maximum(_mm(h, dw2[...]) + db2[...], 0.0)
    out_o[...] = _mm(h, dw3[...]) + db3[...]


def _row_spec(b, d):
    return pl.BlockSpec((b, d), lambda i: (i, 0))


def _full_spec(shape):
    return pl.BlockSpec(shape, lambda i: tuple(0 for _ in shape))


def _wspecs(ws):
    return [_full_spec(w.shape) for w in ws]


def _enc_node_call(x, ws, n):
    f = pl.pallas_call(
        _enc_node_body,
        grid=(n // BN,),
        in_specs=[_row_spec(BN, x.shape[1])] + _wspecs(ws),
        out_specs=[_row_spec(BN, 128), _row_spec(BN, 64), _row_spec(BN, 64)],
        out_shape=[jax.ShapeDtypeStruct((n, 128), F32),
                   jax.ShapeDtypeStruct((n, 64), F32),
                   jax.ShapeDtypeStruct((n, 64), F32)],
    )
    return f(x, *ws)


def _enc_edge_call(ea, ws, e, off_blocks):
    f = pl.pallas_call(
        _enc_edge_body,
        grid=(e // BE,),
        in_specs=[pl.BlockSpec((BE, ea.shape[1]),
                               lambda i: (i + off_blocks, 0))] + _wspecs(ws),
        out_specs=_row_spec(BE, 128),
        out_shape=jax.ShapeDtypeStruct((e, 128), BF16),
    )
    return f(ea, *ws)


def _edge_mlp_call(edge, gv, ws, e, with_new):
    nblk = e // BE
    n_out = 2 if with_new else 1
    f = pl.pallas_call(
        functools.partial(_edge_mlp_body, with_new),
        grid=(nblk,),
        in_specs=[_row_spec(BE, 128), _row_spec(BE, 128)] + _wspecs(ws),
        out_specs=[_row_spec(BE, 128)] * n_out,
        out_shape=[jax.ShapeDtypeStruct((e, 128), F32),
                   jax.ShapeDtypeStruct((e, 128), BF16)][:n_out],
    )
    return f(edge, gv, *ws)


def _node_mlp_call(node, a0, a1, ws, n, with_proj):
    nblk = n // BN
    n_out = 3 if with_proj else 1
    f = pl.pallas_call(
        functools.partial(_node_mlp_body, with_proj),
        grid=(nblk,),
        in_specs=[_row_spec(BN, 128)] * 3 + _wspecs(ws),
        out_specs=[_row_spec(BN, 128)] + [_row_spec(BN, 64)] * (n_out - 1),
        out_shape=[jax.ShapeDtypeStruct((n, 128), F32)] +
                  [jax.ShapeDtypeStruct((n, 64), F32)] * (n_out - 1),
    )
    return f(node, a0, a1, *ws)


def _knn_call(posf, pcT, node, xc, ws, n, n_coarse):
    cp = pcT.shape[1]
    f = pl.pallas_call(
        functools.partial(_knn_body, n_coarse),
        grid=(n // BF,),
        in_specs=[_row_spec(BF, 8), _full_spec((8, cp)),
                  _row_spec(BF, 128), _full_spec((cp, 128))] + _wspecs(ws),
        out_specs=_row_spec(BF, 128),
        out_shape=jax.ShapeDtypeStruct((n, 128), F32),
    )
    return f(posf, pcT, node, xc, *ws)


def _sc_gather(table, idx, chunk):
    b = idx.shape[0]
    d = table.shape[1]
    bpw = b // NWORK
    nchunks = bpw // chunk
    assert bpw % chunk == 0 and chunk % 8 == 0 and b % NWORK == 0
    mesh = plsc.VectorSubcoreMesh(core_axis_name="c", subcore_axis_name="s")

    @functools.partial(
        pl.kernel,
        out_type=jax.ShapeDtypeStruct((b, d), table.dtype),
        mesh=mesh,
        compiler_params=pltpu.CompilerParams(use_tc_tiling_on_sc=False),
        scratch_types=[
            pltpu.VMEM((chunk,), jnp.int32),
            pltpu.VMEM((chunk, d), table.dtype),
            pltpu.SemaphoreType.DMA,
        ],
    )
    def k(table_h, idx_h, out_h, idx_v, rows_v, sem):
        wid = lax.axis_index("s") * NCORES + lax.axis_index("c")
        base0 = wid * bpw

        def body(c, carry):
            base = base0 + c * chunk
            pltpu.sync_copy(idx_h.at[pl.ds(base, chunk)], idx_v)
            pltpu.async_copy(table_h.at[idx_v], rows_v, sem).wait()
            pltpu.sync_copy(rows_v, out_h.at[pl.ds(base, chunk)])
            return carry

        lax.fori_loop(0, nchunks, body, 0)

    return k(table, idx)


def _coop_rows(sid, rows, fn):
    q = (-(-rows // NSUB) + 7) // 8 * 8
    q_last = rows - (NSUB - 1) * q
    assert 0 < q_last <= q and q_last % 8 == 0

    @pl.when(sid < NSUB - 1)
    def _():
        fn(sid * q, q)

    @pl.when(sid == NSUB - 1)
    def _():
        fn((NSUB - 1) * q, q_last)


def _sc_segsum(vals, idx, n_out, chunk):
    e = idx.shape[0]
    d = vals.shape[1]
    dh = d // 2
    ept = e // NSUB
    nchunks = ept // chunk
    assert ept % chunk == 0 and chunk % 8 == 0
    zeros = jnp.zeros((n_out, dh), F32)
    mesh = plsc.VectorSubcoreMesh(core_axis_name="c", subcore_axis_name="s")

    @functools.partial(
        pl.kernel,
        out_type=jax.ShapeDtypeStruct((n_out, d), F32),
        mesh=mesh,
        compiler_params=pltpu.CompilerParams(use_tc_tiling_on_sc=False),
        scratch_types=[
            pltpu.VMEM((chunk,), jnp.int32),
            pltpu.VMEM((chunk, dh), F32),
            pltpu.VMEM_SHARED((n_out, dh), F32),
        ],
    )
    def k(vals_h, idx_h, zer_h, out_h, idx_v, rows_v, shared):
        cid = lax.axis_index("c")
        sid = lax.axis_index("s")

        _coop_rows(sid, n_out,
                   lambda o, s: pltpu.sync_copy(zer_h.at[pl.ds(o, s)],
                                                shared.at[pl.ds(o, s)]))
        plsc.subcore_barrier()

        def run(c0):
            def body(c, carry):
                base = sid * ept + c * chunk
                pltpu.sync_copy(idx_h.at[pl.ds(base, chunk)], idx_v)
                pltpu.sync_copy(
                    vals_h.at[pl.ds(base, chunk), pl.ds(c0, dh)], rows_v)
                pltpu.sync_copy(rows_v, shared.at[idx_v], add=True)
                return carry

            lax.fori_loop(0, nchunks, body, 0)

        def flush(c0):
            _coop_rows(sid, n_out,
                       lambda o, s: pltpu.sync_copy(
                           shared.at[pl.ds(o, s)],
                           out_h.at[pl.ds(o, s), pl.ds(c0, dh)]))

        @pl.when(cid == 0)
        def _():
            run(0)

        @pl.when(cid == 1)
        def _():
            run(dh)

        plsc.subcore_barrier()

        @pl.when(cid == 0)
        def _():
            flush(0)

        @pl.when(cid == 1)
        def _():
            flush(dh)

    return k(vals, idx, zeros)


def _mlp_ws(p, pad_last_to=None):
    (w1, b1), (w2, b2), (w3, b3) = p["layers"]
    g, be = p["ln"]
    return [w1, b1.reshape(1, -1), w2, b2.reshape(1, -1), w3,
            b3.reshape(1, -1), g.reshape(1, -1), be.reshape(1, -1)]


def kernel(x, edge_index, edge_attr, pos, coarse_idx, params):
    n = x.shape[0]
    e = edge_attr.shape[0]
    n_coarse = coarse_idx.shape[0]
    src = edge_index[0].astype(jnp.int32)
    dst = edge_index[1].astype(jnp.int32)

    pe = params["proc_edge"]
    pe_w1 = pe["layers"][0][0]
    w1a, w1b, w1c = pe_w1[:128], pe_w1[128:256], pe_w1[256:384]
    proj_ws = [w1b[:, :64], w1b[:, 64:], w1c[:, :64], w1c[:, 64:]]
    i64 = jnp.eye(64, dtype=BF16)
    z64 = jnp.zeros((128, 64), BF16)
    ii = jnp.concatenate([i64, i64], axis=0)
    m1 = jnp.concatenate([ii, z64], axis=1)
    m2 = jnp.concatenate([z64, ii], axis=1)
    pe_ws = [w1a.astype(BF16), m1, m2, pe["layers"][0][1].reshape(1, -1),
             pe["layers"][1][0].astype(BF16),
             pe["layers"][1][1].reshape(1, -1),
             pe["layers"][2][0].astype(BF16),
             pe["layers"][2][1].reshape(1, -1),
             pe["ln"][0].reshape(1, -1), pe["ln"][1].reshape(1, -1)]

    pn = params["proc_node"]
    pn_w1 = pn["layers"][0][0]
    v1a, v1b = pn_w1[:128], pn_w1[128:]
    pn_ws = [v1a, v1b, pn["layers"][0][1].reshape(1, -1),
             pn["layers"][1][0], pn["layers"][1][1].reshape(1, -1),
             pn["layers"][2][0], pn["layers"][2][1].reshape(1, -1),
             pn["ln"][0].reshape(1, -1), pn["ln"][1].reshape(1, -1),
             ] + proj_ws

    node, ps, pd_ = _enc_node_call(x, _mlp_ws(params["enc_node"]) + proj_ws, n)
    ee_ws = _mlp_ws(params["enc_edge"])
    for i in (0, 2, 4):
        ee_ws[i] = ee_ws[i].astype(BF16)

    e2 = e // 2
    halves = []
    for h in range(2):
        sl = slice(h * e2, (h + 1) * e2)
        halves.append({
            "idx_cat": jnp.stack([src[sl], dst[sl] + n], axis=1).reshape(-1),
            "dst": dst[sl],
            "edge": _enc_edge_call(edge_attr, ee_ws, e2, h * (e2 // BE)),
        })

    for step in range(2):
        last = step == 1
        p_cat = jnp.concatenate([ps, pd_], axis=0)
        gc, upd, agg = [None, None], [None, None], [None, None]
        for h in range(2):
            gc[h] = _sc_gather(p_cat, halves[h]["idx_cat"], 1000
                               ).reshape(e2, 128)
        for h in range(2):
            if last:
                (upd[h],) = _edge_mlp_call(halves[h]["edge"], gc[h], pe_ws,
                                           e2, with_new=False)
            else:
                upd[h], halves[h]["edge"] = _edge_mlp_call(
                    halves[h]["edge"], gc[h], pe_ws, e2, with_new=True)
            agg[h] = _sc_segsum(upd[h], halves[h]["dst"], n, 1000)
        if last:
            (node,) = _node_mlp_call(node, agg[0], agg[1], pn_ws, n,
                                     with_proj=False)
        else:
            node, ps, pd_ = _node_mlp_call(node, agg[0], agg[1], pn_ws, n,
                                           with_proj=True)

    cpad = ((n_coarse + 8 * NWORK - 1) // (8 * NWORK)) * (8 * NWORK)
    cidx = jnp.concatenate(
        [coarse_idx.astype(jnp.int32),
         jnp.zeros((cpad - n_coarse,), jnp.int32)])
    xc = _sc_gather(node, cidx, cpad // NWORK)

    pos_c = jnp.take(pos, coarse_idx, axis=0)
    pcT = jnp.zeros((8, cpad), F32).at[:3, :n_coarse].set(pos_c.T)
    posf = jnp.pad(pos, ((0, 0), (0, 5)))

    po_ws = _mlp_ws(params["post"])
    (dw1, db1), (dw2, db2), (dw3, db3) = params["dec"]["layers"]
    out_dim = dw3.shape[1]
    dw3p = jnp.zeros((128, 128), F32).at[:, :out_dim].set(dw3)
    db3p = jnp.zeros((1, 128), F32).at[0, :out_dim].set(db3)
    de_ws = [dw1, db1.reshape(1, -1), dw2, db2.reshape(1, -1), dw3p, db3p]

    outp = _knn_call(posf, pcT, node, xc, po_ws + de_ws, n, n_coarse)
    return outp[:, :out_dim]


# speedup vs baseline: 1.2795x; 1.0026x over previous
import functools

import jax
import jax.numpy as jnp
from jax import lax
from jax.experimental import pallas as pl
from jax.experimental.pallas import tpu as pltpu
from jax.experimental.pallas import tpu_sc as plsc

F32 = jnp.float32
BF16 = jnp.bfloat16
EPS = 1e-5
NCORES, NSUB = 2, 16
NWORK = NCORES * NSUB

BN = 10000
BE = 4000
BF = 1000


def _pack2(pa, pb):
    ua = lax.bitcast_convert_type(pa, jnp.uint32)
    ub = lax.bitcast_convert_type(pb, jnp.uint32)
    ra = (ua + ((ua >> 16) & 1) + 0x7FFF) >> 16
    rb = (ub + ((ub >> 16) & 1) + 0x7FFF) & jnp.uint32(0xFFFF0000)
    return lax.bitcast_convert_type(ra | rb, F32)


def _unpack2(g):
    u = lax.bitcast_convert_type(g, jnp.uint32)
    lo = lax.bitcast_convert_type(u << 16, F32)
    hi = lax.bitcast_convert_type(u & jnp.uint32(0xFFFF0000), F32)
    return lo, hi


def _ln(h, g, be):
    m = jnp.mean(h, axis=-1, keepdims=True)
    v = jnp.mean((h - m) ** 2, axis=-1, keepdims=True)
    return (h - m) * lax.rsqrt(v + EPS) * g + be


def _mm(a, b):
    return jnp.dot(a, b, preferred_element_type=F32)


def _proj_pack(node, ws_lo, ws_hi):
    return _pack2(_mm(node, ws_lo[...]), _mm(node, ws_hi[...]))


def _enc_node_body(x_r, w1, b1, w2, b2, w3, b3, g, be,
                   ws_lo, ws_hi, wd_lo, wd_hi, node_o, ps_o, pd_o):
    h = jnp.maximum(_mm(x_r[...], w1[...]) + b1[...], 0.0)
    h = jnp.maximum(_mm(h, w2[...]) + b2[...], 0.0)
    h = _mm(h, w3[...]) + b3[...]
    node = _ln(h, g[...], be[...])
    node_o[...] = node
    ps_o[...] = _proj_pack(node, ws_lo, ws_hi)
    pd_o[...] = _proj_pack(node, wd_lo, wd_hi)


def _enc_edge_body(x_r, w1, b1, w2, b2, w3, b3, g, be, edge_o):
    h = jnp.maximum(_mm(x_r[...].astype(BF16), w1[...]) + b1[...], 0.0)
    h = jnp.maximum(_mm(h.astype(BF16), w2[...]) + b2[...], 0.0)
    h = _mm(h.astype(BF16), w3[...]) + b3[...]
    edge_o[...] = _ln(h, g[...], be[...]).astype(BF16)


def _edge_mlp_body(with_new, edge_r, g_r, w1a, m1, m2, b1, w2, b2, w3, b3,
                   g, be, *outs):
    u = lax.bitcast_convert_type(g_r[...], jnp.uint32)
    lo = lax.bitcast_convert_type(u << 16, F32).astype(BF16)
    hi = lax.bitcast_convert_type(u & jnp.uint32(0xFFFF0000), F32).astype(BF16)
    h1 = jnp.maximum(_mm(edge_r[...], w1a[...]) + _mm(lo, m1[...])
                     + _mm(hi, m2[...]) + b1[...], 0.0)
    h2 = jnp.maximum(_mm(h1.astype(BF16), w2[...]) + b2[...], 0.0)
    h3 = _mm(h2.astype(BF16), w3[...]) + b3[...]
    upd = _ln(h3, g[...], be[...])
    outs[0][...] = upd
    if with_new:
        outs[1][...] = (edge_r[...].astype(F32) + upd).astype(BF16)


def _node_mlp_body(with_proj, node_r, a0_r, a1_r, v1a, v1b, b1, v2, b2,
                   v3, b3, g, be, ws_lo, ws_hi, wd_lo, wd_hi, *outs):
    agg = a0_r[...] + a1_r[...]
    h = jnp.maximum(_mm(node_r[...], v1a[...]) + _mm(agg, v1b[...])
                    + b1[...], 0.0)
    h = jnp.maximum(_mm(h, v2[...]) + b2[...], 0.0)
    h = _mm(h, v3[...]) + b3[...]
    node = node_r[...] + _ln(h, g[...], be[...])
    outs[0][...] = node
    if with_proj:
        outs[1][...] = _proj_pack(node, ws_lo, ws_hi)
        outs[2][...] = _proj_pack(node, wd_lo, wd_hi)


def _knn_body(n_coarse, pf_r, pcT_r, node_r, xc_r,
              pw1, pb1, pw2, pb2, pw3, pb3, pg, pbe,
              dw1, db1, dw2, db2, dw3, db3, out_o):
    bf = node_r.shape[0]
    cp = pcT_r.shape[1]
    inf = jnp.float32(jnp.inf)
    d2 = jnp.zeros((bf, cp), F32)
    for d in range(3):
        diff = pf_r[:, d:d + 1] - pcT_r[d, :][None, :]
        d2 = d2 + diff * diff
    colj = lax.broadcasted_iota(jnp.int32, (bf, cp), 1)
    d2 = jnp.where(colj >= n_coarse, inf, d2)
    wacc = jnp.zeros((bf, cp), F32)
    wsum = jnp.zeros((bf, 1), F32)
    for _k in range(3):
        m = jnp.min(d2, axis=1, keepdims=True)
        idx = jnp.min(jnp.where(d2 == m, colj, cp), axis=1, keepdims=True)
        sel = colj == idx
        w = 1.0 / jnp.maximum(m, 1e-16)
        wacc = wacc + jnp.where(sel, w, 0.0)
        wsum = wsum + w
        d2 = jnp.where(sel, inf, d2)
    interp = _mm(wacc, xc_r[...]) / wsum
    nf = node_r[...] + interp
    h = jnp.maximum(_mm(nf, pw1[...]) + pb1[...], 0.0)
    h = jnp.maximum(_mm(h, pw2[...]) + pb2[...], 0.0)
    h = _mm(h, pw3[...]) + pb3[...]
    h = _ln(h, pg[...], pbe[...])
    h = jnp.maximum(_mm(h, dw1[...]) + db1[...], 0.0)
    h = jnp.maximum(_mm(h, dw2[...]) + db2[...], 0.0)
    out_o[...] = _mm(h, dw3[...]) + db3[...]


def _row_spec(b, d):
    return pl.BlockSpec((b, d), lambda i: (i, 0))


def _full_spec(shape):
    return pl.BlockSpec(shape, lambda i: tuple(0 for _ in shape))


def _wspecs(ws):
    return [_full_spec(w.shape) for w in ws]


def _enc_node_call(x, ws, n):
    f = pl.pallas_call(
        _enc_node_body,
        grid=(n // BN,),
        in_specs=[_row_spec(BN, x.shape[1])] + _wspecs(ws),
        out_specs=[_row_spec(BN, 128), _row_spec(BN, 64), _row_spec(BN, 64)],
        out_shape=[jax.ShapeDtypeStruct((n, 128), F32),
                   jax.ShapeDtypeStruct((n, 64), F32),
                   jax.ShapeDtypeStruct((n, 64), F32)],
    )
    return f(x, *ws)


def _enc_edge_call(ea, ws, e, off_blocks):
    f = pl.pallas_call(
        _enc_edge_body,
        grid=(e // BE,),
        in_specs=[pl.BlockSpec((BE, ea.shape[1]),
                               lambda i: (i + off_blocks, 0))] + _wspecs(ws),
        out_specs=_row_spec(BE, 128),
        out_shape=jax.ShapeDtypeStruct((e, 128), BF16),
    )
    return f(ea, *ws)


def _edge_mlp_call(edge, gv, ws, e, with_new):
    nblk = e // BE
    n_out = 2 if with_new else 1
    f = pl.pallas_call(
        functools.partial(_edge_mlp_body, with_new),
        grid=(nblk,),
        in_specs=[_row_spec(BE, 128), _row_spec(BE, 128)] + _wspecs(ws),
        out_specs=[_row_spec(BE, 128)] * n_out,
        out_shape=[jax.ShapeDtypeStruct((e, 128), F32),
                   jax.ShapeDtypeStruct((e, 128), BF16)][:n_out],
    )
    return f(edge, gv, *ws)


def _node_mlp_call(node, a0, a1, ws, n, with_proj):
    nblk = n // BN
    n_out = 3 if with_proj else 1
    f = pl.pallas_call(
        functools.partial(_node_mlp_body, with_proj),
        grid=(nblk,),
        in_specs=[_row_spec(BN, 128)] * 3 + _wspecs(ws),
        out_specs=[_row_spec(BN, 128)] + [_row_spec(BN, 64)] * (n_out - 1),
        out_shape=[jax.ShapeDtypeStruct((n, 128), F32)] +
                  [jax.ShapeDtypeStruct((n, 64), F32)] * (n_out - 1),
    )
    return f(node, a0, a1, *ws)


def _knn_call(posf, pcT, node, xc, ws, n, n_coarse):
    cp = pcT.shape[1]
    f = pl.pallas_call(
        functools.partial(_knn_body, n_coarse),
        grid=(n // BF,),
        in_specs=[_row_spec(BF, 8), _full_spec((8, cp)),
                  _row_spec(BF, 128), _full_spec((cp, 128))] + _wspecs(ws),
        out_specs=_row_spec(BF, 128),
        out_shape=jax.ShapeDtypeStruct((n, 128), F32),
    )
    return f(posf, pcT, node, xc, *ws)


def _sc_gather(table, idx, chunk):
    b = idx.shape[0]
    d = table.shape[1]
    bpw = b // NWORK
    nchunks = bpw // chunk
    assert bpw % chunk == 0 and chunk % 8 == 0 and b % NWORK == 0
    mesh = plsc.VectorSubcoreMesh(core_axis_name="c", subcore_axis_name="s")

    @functools.partial(
        pl.kernel,
        out_type=jax.ShapeDtypeStruct((b, d), table.dtype),
        mesh=mesh,
        compiler_params=pltpu.CompilerParams(use_tc_tiling_on_sc=False),
        scratch_types=[
            pltpu.VMEM((chunk,), jnp.int32),
            pltpu.VMEM((chunk, d), table.dtype),
            pltpu.SemaphoreType.DMA,
        ],
    )
    def k(table_h, idx_h, out_h, idx_v, rows_v, sem):
        wid = lax.axis_index("s") * NCORES + lax.axis_index("c")
        base0 = wid * bpw

        def body(c, carry):
            base = base0 + c * chunk
            pltpu.sync_copy(idx_h.at[pl.ds(base, chunk)], idx_v)
            pltpu.async_copy(table_h.at[idx_v], rows_v, sem).wait()
            pltpu.sync_copy(rows_v, out_h.at[pl.ds(base, chunk)])
            return carry

        lax.fori_loop(0, nchunks, body, 0)

    return k(table, idx)


def _coop_rows(sid, rows, fn):
    q = (-(-rows // NSUB) + 7) // 8 * 8
    q_last = rows - (NSUB - 1) * q
    assert 0 < q_last <= q and q_last % 8 == 0

    @pl.when(sid < NSUB - 1)
    def _():
        fn(sid * q, q)

    @pl.when(sid == NSUB - 1)
    def _():
        fn((NSUB - 1) * q, q_last)


def _sc_segsum(vals, idx, n_out, chunk):
    e = idx.shape[0]
    d = vals.shape[1]
    dh = d // 2
    ept = e // NSUB
    nchunks = ept // chunk
    assert ept % chunk == 0 and chunk % 8 == 0
    zeros = jnp.zeros((n_out, dh), F32)
    mesh = plsc.VectorSubcoreMesh(core_axis_name="c", subcore_axis_name="s")

    @functools.partial(
        pl.kernel,
        out_type=jax.ShapeDtypeStruct((n_out, d), F32),
        mesh=mesh,
        compiler_params=pltpu.CompilerParams(use_tc_tiling_on_sc=False),
        scratch_types=[
            pltpu.VMEM((chunk,), jnp.int32),
            pltpu.VMEM((chunk, dh), F32),
            pltpu.VMEM_SHARED((n_out, dh), F32),
        ],
    )
    def k(vals_h, idx_h, zer_h, out_h, idx_v, rows_v, shared):
        cid = lax.axis_index("c")
        sid = lax.axis_index("s")

        _coop_rows(sid, n_out,
                   lambda o, s: pltpu.sync_copy(zer_h.at[pl.ds(o, s)],
                                                shared.at[pl.ds(o, s)]))
        plsc.subcore_barrier()

        def run(c0):
            def body(c, carry):
                base = sid * ept + c * chunk
                pltpu.sync_copy(idx_h.at[pl.ds(base, chunk)], idx_v)
                pltpu.sync_copy(
                    vals_h.at[pl.ds(base, chunk), pl.ds(c0, dh)], rows_v)
                pltpu.sync_copy(rows_v, shared.at[idx_v], add=True)
                return carry

            lax.fori_loop(0, nchunks, body, 0)

        def flush(c0):
            _coop_rows(sid, n_out,
                       lambda o, s: pltpu.sync_copy(
                           shared.at[pl.ds(o, s)],
                           out_h.at[pl.ds(o, s), pl.ds(c0, dh)]))

        @pl.when(cid == 0)
        def _():
            run(0)

        @pl.when(cid == 1)
        def _():
            run(dh)

        plsc.subcore_barrier()

        @pl.when(cid == 0)
        def _():
            flush(0)

        @pl.when(cid == 1)
        def _():
            flush(dh)

    return k(vals, idx, zeros)


def _mlp_ws(p, pad_last_to=None):
    (w1, b1), (w2, b2), (w3, b3) = p["layers"]
    g, be = p["ln"]
    return [w1, b1.reshape(1, -1), w2, b2.reshape(1, -1), w3,
            b3.reshape(1, -1), g.reshape(1, -1), be.reshape(1, -1)]


def kernel(x, edge_index, edge_attr, pos, coarse_idx, params):
    n = x.shape[0]
    e = edge_attr.shape[0]
    n_coarse = coarse_idx.shape[0]
    src = edge_index[0].astype(jnp.int32)
    dst = edge_index[1].astype(jnp.int32)

    pe = params["proc_edge"]
    pe_w1 = pe["layers"][0][0]
    w1a, w1b, w1c = pe_w1[:128], pe_w1[128:256], pe_w1[256:384]
    proj_ws = [w1b[:, :64], w1b[:, 64:], w1c[:, :64], w1c[:, 64:]]
    i64 = jnp.eye(64, dtype=BF16)
    z64 = jnp.zeros((128, 64), BF16)
    ii = jnp.concatenate([i64, i64], axis=0)
    m1 = jnp.concatenate([ii, z64], axis=1)
    m2 = jnp.concatenate([z64, ii], axis=1)
    pe_ws = [w1a.astype(BF16), m1, m2, pe["layers"][0][1].reshape(1, -1),
             pe["layers"][1][0].astype(BF16),
             pe["layers"][1][1].reshape(1, -1),
             pe["layers"][2][0].astype(BF16),
             pe["layers"][2][1].reshape(1, -1),
             pe["ln"][0].reshape(1, -1), pe["ln"][1].reshape(1, -1)]

    pn = params["proc_node"]
    pn_w1 = pn["layers"][0][0]
    v1a, v1b = pn_w1[:128], pn_w1[128:]
    pn_ws = [v1a, v1b, pn["layers"][0][1].reshape(1, -1),
             pn["layers"][1][0], pn["layers"][1][1].reshape(1, -1),
             pn["layers"][2][0], pn["layers"][2][1].reshape(1, -1),
             pn["ln"][0].reshape(1, -1), pn["ln"][1].reshape(1, -1),
             ] + proj_ws

    node, ps, pd_ = _enc_node_call(x, _mlp_ws(params["enc_node"]) + proj_ws, n)
    ee_ws = _mlp_ws(params["enc_edge"])
    for i in (0, 2, 4):
        ee_ws[i] = ee_ws[i].astype(BF16)

    e2 = e // 2
    halves = []
    for h in range(2):
        sl = slice(h * e2, (h + 1) * e2)
        halves.append({
            "idx_cat": jnp.stack([src[sl], dst[sl] + n], axis=1).reshape(-1),
            "dst": dst[sl],
            "edge": _enc_edge_call(edge_attr, ee_ws, e2, h * (e2 // BE)),
        })

    for step in range(2):
        last = step == 1
        p_cat = jnp.concatenate([ps, pd_], axis=0)
        gc, upd, agg = [None, None], [None, None], [None, None]
        for h in range(2):
            gc[h] = _sc_gather(p_cat, halves[h]["idx_cat"], 1000
                               ).reshape(e2, 128)
        for h in range(2):
            if last:
                (upd[h],) = _edge_mlp_call(halves[h]["edge"], gc[h], pe_ws,
                                           e2, with_new=False)
            else:
                upd[h], halves[h]["edge"] = _edge_mlp_call(
                    halves[h]["edge"], gc[h], pe_ws, e2, with_new=True)
            agg[h] = _sc_segsum(upd[h], halves[h]["dst"], n, 1000)
        if last:
            (node,) = _node_mlp_call(node, agg[0], agg[1], pn_ws, n,
                                     with_proj=False)
        else:
            node, ps, pd_ = _node_mlp_call(node, agg[0], agg[1], pn_ws, n,
                                           with_proj=True)

    cpad = ((n_coarse + 8 * NWORK - 1) // (8 * NWORK)) * (8 * NWORK)
    cidx = jnp.concatenate(
        [coarse_idx.astype(jnp.int32),
         jnp.zeros((cpad - n_coarse,), jnp.int32)])
    xc = _sc_gather(node, cidx, cpad // NWORK)

    pos_c = jnp.take(pos, coarse_idx, axis=0)
    pcT = jnp.zeros((8, cpad), F32).at[:3, :n_coarse].set(pos_c.T)
    posf = jnp.pad(pos, ((0, 0), (0, 5)))

    po_ws = _mlp_ws(params["post"])
    (dw1, db1), (dw2, db2), (dw3, db3) = params["dec"]["layers"]
    out_dim = dw3.shape[1]
    dw3p = jnp.zeros((128, 128), F32).at[:, :out_dim].set(dw3)
    db3p = jnp.zeros((1, 128), F32).at[0, :out_dim].set(db3)
    de_ws = [dw1, db1.reshape(1, -1), dw2, db2.reshape(1, -1), dw3p, db3p]

    outp = _knn_call(posf, pcT, node, xc, po_ws + de_ws, n, n_coarse)
    return outp[:, :out_dim]
